# Initial kernel scaffold; baseline (speedup 1.0000x reference)
import jax
import jax.numpy as jnp
from jax.experimental import pallas as pl


def kernel(x, edge_index, edge_attr, W, b):
    raise NotImplementedError("write your pallas kernel here")


# speedup vs baseline: 8.9997x; 8.9997x over previous
import functools

import jax
import jax.numpy as jnp
from jax import lax
from jax.experimental import pallas as pl
from jax.experimental.pallas import tpu as pltpu
from jax.experimental.pallas import tpu_sc as plsc

NC = 2
NS = 16
NW = NC * NS
LN = 16
CB = 128


def _mesh():
    return plsc.VectorSubcoreMesh(core_axis_name="c", subcore_axis_name="s")


def _zero16():
    return jnp.zeros((LN,), jnp.float32)


def _deg_call(n_acc, ch):
    npt = n_acc // NS

    @functools.partial(
        pl.kernel,
        out_type=jax.ShapeDtypeStruct((NC * n_acc,), jnp.float32),
        mesh=_mesh(),
        scratch_types=[
            pltpu.VMEM((ch, CB), jnp.int32),
            pltpu.VMEM((ch, CB), jnp.float32),
            pltpu.VMEM((npt,), jnp.float32),
            pltpu.VMEM_SHARED((n_acc,), jnp.float32),
        ],
    )
    def k(col_hbm, ew_hbm, out_hbm, col_v, ew_v, zb, acc):
        cid = lax.axis_index("c")
        sid = lax.axis_index("s")
        wid = sid * NC + cid
        pltpu.sync_copy(col_hbm.at[pl.ds(wid * ch, ch)], col_v)
        pltpu.sync_copy(ew_hbm.at[pl.ds(wid * ch, ch)], ew_v)
        z16 = _zero16()

        def zb_body(i, carry):
            zb[pl.ds(i * LN, LN)] = z16
            return carry

        lax.fori_loop(0, npt // LN, zb_body, 0)
        pltpu.sync_copy(zb, acc.at[pl.ds(sid * npt, npt)])
        plsc.subcore_barrier()

        def ch_body(i, carry):
            pltpu.sync_copy(ew_v.at[i], acc.at[col_v.at[i]], add=True)
            return carry

        lax.fori_loop(0, ch, ch_body, 0)
        plsc.subcore_barrier()
        pltpu.sync_copy(acc.at[pl.ds(sid * npt, npt)],
                        out_hbm.at[pl.ds(cid * n_acc + sid * npt, npt)])

    return k


def _prop_call(n_acc, ch, d):
    npt = n_acc // NS
    nko = npt // CB

    @functools.partial(
        pl.kernel,
        out_type=jax.ShapeDtypeStruct((NC, n_acc, d), jnp.float32),
        mesh=_mesh(),
        scratch_types=[
            pltpu.VMEM((ch, CB), jnp.int32),
            pltpu.VMEM((ch, CB), jnp.int32),
            pltpu.VMEM((ch, CB), jnp.float32),
            pltpu.VMEM((CB, d), jnp.float32),
            pltpu.VMEM_SHARED((n_acc, d), jnp.float32),
        ],
    )
    def k(row_hbm, col_hbm, ew_hbm, z_hbm, out_hbm, row_v, col_v, ew_v, buf, acc):
        cid = lax.axis_index("c")
        sid = lax.axis_index("s")
        wid = sid * NC + cid
        pltpu.sync_copy(row_hbm.at[pl.ds(wid * ch, ch)], row_v)
        pltpu.sync_copy(col_hbm.at[pl.ds(wid * ch, ch)], col_v)
        pltpu.sync_copy(ew_hbm.at[pl.ds(wid * ch, ch)], ew_v)
        z16 = _zero16()

        def zrow(r, carry):
            for j in range(d // LN):
                buf[r, pl.ds(j * LN, LN)] = z16
            return carry

        lax.fori_loop(0, CB, zrow, 0)
        for kk in range(nko):
            pltpu.sync_copy(buf, acc.at[pl.ds(sid * npt + kk * CB, CB)])
        plsc.subcore_barrier()

        def ch_body(i, carry):
            pltpu.sync_copy(z_hbm.at[row_v.at[i]], buf)

            def g_body(g, gcarry):
                wv = ew_v[i, pl.ds(g * LN, LN)]
                base = g * LN
                for t in range(LN):
                    s = wv[t]
                    for j in range(d // LN):
                        sl = pl.ds(j * LN, LN)
                        buf[base + t, sl] = buf[base + t, sl] * s
                return gcarry

            lax.fori_loop(0, CB // LN, g_body, 0)
            pltpu.sync_copy(buf, acc.at[col_v.at[i]], add=True)
            return carry

        lax.fori_loop(0, ch, ch_body, 0)
        plsc.subcore_barrier()
        for kk in range(nko):
            sl = pl.ds(sid * npt + kk * CB, CB)
            pltpu.sync_copy(acc.at[sl], out_hbm.at[cid, sl])

    return k


def _scale_x(d0, d1, x):
    n, d = x.shape
    br = 1000

    def body(d0_r, d1_r, x_r, z_r, dinv_r):
        deg = d0_r[...] + d1_r[...] + 1.0
        dinv = jnp.where(deg > 0.0, lax.rsqrt(deg), 0.0)
        dinv_r[...] = dinv
        z_r[...] = x_r[...] * dinv

    return pl.pallas_call(
        body,
        grid=(n // br,),
        in_specs=[pl.BlockSpec((br, 1), lambda r: (r, 0)),
                  pl.BlockSpec((br, 1), lambda r: (r, 0)),
                  pl.BlockSpec((br, d), lambda r: (r, 0))],
        out_specs=[pl.BlockSpec((br, d), lambda r: (r, 0)),
                   pl.BlockSpec((br, 1), lambda r: (r, 0))],
        out_shape=[jax.ShapeDtypeStruct((n, d), jnp.float32),
                   jax.ShapeDtypeStruct((n, 1), jnp.float32)],
    )(d0, d1, x)


def _mid_rescale(u0, u1, z0, dinv):
    n, d = z0.shape
    br = 1000

    def body(u0_r, u1_r, z_r, dinv_r, o_r):
        di = dinv_r[...]
        o_r[...] = (u0_r[...] + u1_r[...] + z_r[...]) * (di * di)

    return pl.pallas_call(
        body,
        grid=(n // br,),
        in_specs=[pl.BlockSpec((br, d), lambda r: (r, 0)),
                  pl.BlockSpec((br, d), lambda r: (r, 0)),
                  pl.BlockSpec((br, d), lambda r: (r, 0)),
                  pl.BlockSpec((br, 1), lambda r: (r, 0))],
        out_specs=pl.BlockSpec((br, d), lambda r: (r, 0)),
        out_shape=jax.ShapeDtypeStruct((n, d), jnp.float32),
    )(u0, u1, z0, dinv)


def _head(u0, u1, z1, dinv, W, b2):
    n, d = z1.shape
    dout = W.shape[0]
    br = 1000

    def body(u0_r, u1_r, z_r, dinv_r, w_r, b_r, o_r):
        h = (u0_r[...] + u1_r[...] + z_r[...]) * dinv_r[...]
        y = lax.dot_general(h, w_r[...], (((1,), (1,)), ((), ())),
                            precision=lax.Precision.HIGHEST)
        y = jnp.maximum(y + b_r[...], 0.0)
        m = jnp.max(y, axis=-1, keepdims=True)
        lse = jnp.log(jnp.sum(jnp.exp(y - m), axis=-1, keepdims=True)) + m
        o_r[...] = y - lse

    return pl.pallas_call(
        body,
        grid=(n // br,),
        in_specs=[pl.BlockSpec((br, d), lambda r: (r, 0)),
                  pl.BlockSpec((br, d), lambda r: (r, 0)),
                  pl.BlockSpec((br, d), lambda r: (r, 0)),
                  pl.BlockSpec((br, 1), lambda r: (r, 0)),
                  pl.BlockSpec((dout, d), lambda r: (0, 0)),
                  pl.BlockSpec((1, dout), lambda r: (0, 0))],
        out_specs=pl.BlockSpec((br, dout), lambda r: (r, 0)),
        out_shape=jax.ShapeDtypeStruct((n, dout), jnp.float32),
    )(u0, u1, z1, dinv, W, b2)


def kernel(x, edge_index, edge_attr, W, b):
    n, d = x.shape
    e = edge_index.shape[1]
    row = edge_index[0].astype(jnp.int32)
    col = edge_index[1].astype(jnp.int32)
    ew = edge_attr.astype(jnp.float32)

    ch = -(-(-(-e // (NW * CB))) // 8) * 8
    e_pad = ch * NW * CB
    pad = e_pad - e
    zi = jnp.zeros((pad,), jnp.int32)
    row_p = jnp.concatenate([row, zi]).reshape(NW * ch, CB)
    col_p = jnp.concatenate([col, zi]).reshape(NW * ch, CB)
    ew_p = jnp.concatenate([ew, jnp.zeros((pad,), jnp.float32)]).reshape(NW * ch, CB)

    npt = -(-n // (NS * CB)) * CB
    n_acc = npt * NS

    degp = _deg_call(n_acc, ch)(col_p, ew_p).reshape(NC, n_acc)
    d0 = degp[0, :n].reshape(n, 1)
    d1 = degp[1, :n].reshape(n, 1)
    z0, dinv = _scale_x(d0, d1, x)

    up1 = _prop_call(n_acc, ch, d)(row_p, col_p, ew_p, z0)
    z1 = _mid_rescale(up1[0, :n], up1[1, :n], z0, dinv)

    up2 = _prop_call(n_acc, ch, d)(row_p, col_p, ew_p, z1)
    return _head(up2[0, :n], up2[1, :n], z1, dinv, W, b.reshape(1, -1))
